# baseline (speedup 1.0000x reference)
import jax
import jax.numpy as jnp
from jax import lax
from jax.experimental import pallas as pl
from jax.experimental.pallas import tpu as pltpu
from jax.experimental.pallas import tpu_sc as plsc

B = 128
Q = 900
C = 91
N = Q * C
NPAD = 81920
NVEC = NPAD // 16
NG = NPAD // 128
TAIL0 = (N // 16) * 16
NREAL = N - TAIL0
K_SEL = 120
NC = 2
NS = 16
NW = NC * NS
IMG_PER_W = B // NW
CAND = 160
NCV = CAND // 16
CAPA = 8192
NSAMP = 80
K_G = 12


def _topk_body(prob_hbm, boxes_hbm, scale_hbm,
               scores_hbm, labels_hbm, oboxes_hbm,
               pvm, bxvm, scvm, subhist, cand, aibuf, avbuf,
               outsc, outidx, outlb, outbx):
    wid = lax.axis_index("s") * NC + lax.axis_index("c")
    iota = lax.iota(jnp.int32, 16)
    ones_i = jnp.ones((16,), jnp.int32)
    zero_i = jnp.zeros((16,), jnp.int32)
    k120 = jnp.full((16,), K_SEL, jnp.int32)
    kg = jnp.full((16,), K_G, jnp.int32)

    def zero_hist():
        def zh(z, _):
            subhist[pl.ds(z * 16, 16)] = zero_i
            return 0
        lax.fori_loop(0, 256, zh, 0, unroll=8)

    def hist_walk(above_in, target):
        need = target - above_in

        def walk(c2, carry):
            found, bsel, above, csum = carry
            c = 15 - c2
            base = 256 * c
            h = zero_i
            for l in range(16):
                h = h + plsc.load_gather(
                    subhist, [base + jnp.left_shift(iota, 4) + l])
            rh = lax.rev(h, (0,))
            cs = plsc.cumsum(rh)
            tot = jnp.sum(h)
            contains = jnp.logical_and(jnp.logical_not(found),
                                       csum + tot >= need)
            mvec = (csum + cs) >= need
            r = plsc.all_reduce_ffs(mvec)
            b_here = 16 * c + 15 - r
            above_here = above + csum + jnp.sum(jnp.where(iota < r, rh, 0))
            found2 = jnp.logical_or(found, contains)
            bsel2 = jnp.where(contains, b_here, bsel)
            above2 = jnp.where(contains, above_here, above)
            return found2, bsel2, above2, csum + tot
        found0 = jnp.zeros((16,), jnp.bool_)
        _, bsel, above_out, _ = lax.fori_loop(
            0, 16, walk, (found0, zero_i, above_in, zero_i))
        return bsel, above_out

    def per_image(t, _):
        img = wid * IMG_PER_W + t

        pltpu.sync_copy(prob_hbm.at[img], pvm.at[pl.ds(0, N)])
        pltpu.sync_copy(boxes_hbm.at[img], bxvm)
        pltpu.sync_copy(scale_hbm.at[img], scvm.at[pl.ds(0, 4)])
        tl = pvm[pl.ds(TAIL0, 16)]
        pvm[pl.ds(TAIL0, 16)] = jnp.where(iota < NREAL, tl, 0.0)
        pvm[pl.ds(TAIL0 + 16, 16)] = jnp.zeros((16,), jnp.float32)
        for v_ in range(NCV):
            cand[pl.ds(16 * v_, 16)] = jnp.full((16,), N, jnp.int32)

        zero_hist()

        def samp1(i, _):
            v = pvm[pl.ds(i * 1024, 16)]
            k = lax.bitcast_convert_type(v, jnp.int32)
            avbuf[pl.ds(i * 16, 16)] = k
            slot = jnp.left_shift(lax.shift_right_logical(k, 24), 4) + iota
            plsc.addupdate_scatter(subhist, [slot], ones_i)
            return 0
        lax.fori_loop(0, NSAMP, samp1, 0, unroll=8)
        b1s, a1s = hist_walk(zero_i, kg)
        zero_hist()

        def samp2(i, _):
            k = avbuf[pl.ds(i * 16, 16)]
            act = lax.shift_right_logical(k, 24) == b1s
            slot = jnp.left_shift(
                jnp.bitwise_and(lax.shift_right_logical(k, 16), 255), 4) + iota
            plsc.addupdate_scatter(subhist, [slot], ones_i, mask=act)
            return 0
        lax.fori_loop(0, NSAMP, samp2, 0, unroll=8)
        b2s, _ = hist_walk(a1s, kg)
        tg = jnp.left_shift(jnp.left_shift(b1s, 8) | b2s, 16)

        def scang(g, aoff):
            base = g * 128
            ks = []
            ms = []
            anym = None
            for u in range(8):
                v = pvm[pl.ds(base + u * 16, 16)]
                k = lax.bitcast_convert_type(v, jnp.int32)
                m = k >= tg
                ks.append(k)
                ms.append(m)
                anym = m if anym is None else jnp.logical_or(anym, m)
            pred = jnp.max(plsc.all_reduce_population_count(anym)) > 0

            def hit(aoff):
                for u in range(8):
                    m0 = ms[u]
                    m = jnp.logical_and(m0, aoff < CAPA - 16)
                    mi = m.astype(jnp.int32)
                    exc = plsc.cumsum(mi) - mi
                    plsc.store_scatter(aibuf, [aoff + exc],
                                       base + u * 16 + iota, mask=m)
                    plsc.store_scatter(avbuf, [aoff + exc], ks[u], mask=m)
                    aoff = aoff + plsc.all_reduce_population_count(m0)
                return aoff

            def miss(aoff):
                return aoff
            return lax.cond(pred, hit, miss, aoff)
        acnt = lax.fori_loop(0, NG, scang, zero_i)
        acnt_s = jnp.max(acnt)

        def fastp(_):
            nv = lax.div(acnt_s + 15, jnp.int32(16))

            def buf_pass(sh, prefix, above):
                zero_hist()

                def sc(i, _, sh=sh, prefix=prefix):
                    av = avbuf[pl.ds(i * 16, 16)]
                    valid = (i * 16 + iota) < acnt
                    if prefix is None:
                        act = valid
                    else:
                        act = jnp.logical_and(
                            valid,
                            lax.shift_right_logical(av, sh + 8) == prefix)
                    slot = jnp.left_shift(
                        jnp.bitwise_and(
                            lax.shift_right_logical(av, sh), 255), 4) + iota
                    plsc.addupdate_scatter(subhist, [slot], ones_i, mask=act)
                    return 0
                lax.fori_loop(0, nv, sc, 0)
                return hist_walk(above, k120)

            b1, a1 = buf_pass(24, None, zero_i)
            b2, a2 = buf_pass(16, b1, a1)
            pre16 = jnp.left_shift(b1, 8) | b2
            b3, a3 = buf_pass(8, pre16, a2)
            pre24 = jnp.left_shift(pre16, 8) | b3
            b4, mG = buf_pass(0, pre24, a3)
            T = jnp.left_shift(pre24, 8) | b4

            def ext(i, offs):
                goff, eoff = offs
                av = avbuf[pl.ds(i * 16, 16)]
                ai = aibuf[pl.ds(i * 16, 16)]
                valid = (i * 16 + iota) < acnt
                gm = jnp.logical_and(valid, av > T)
                em = jnp.logical_and(jnp.logical_and(valid, av == T),
                                     eoff < CAND - 32)
                gmi = gm.astype(jnp.int32)
                emi = em.astype(jnp.int32)
                gexc = plsc.cumsum(gmi) - gmi
                eexc = plsc.cumsum(emi) - emi
                plsc.store_scatter(cand, [goff + gexc], ai, mask=gm)
                plsc.store_scatter(cand, [eoff + eexc], ai, mask=em)
                return (goff + plsc.all_reduce_population_count(gm),
                        eoff + plsc.all_reduce_population_count(em))
            lax.fori_loop(0, nv, ext, (zero_i, mG))
            return jnp.int32(0)

        def slowp(_):
            prefix = zero_i
            above = zero_i
            for pp in range(4):
                sh = 24 - 8 * pp
                zero_hist()

                def sc(i, _, sh=sh, pp=pp, prefix=prefix):
                    v = pvm[pl.ds(i * 16, 16)]
                    k = lax.bitcast_convert_type(v, jnp.int32)
                    slot = jnp.left_shift(
                        jnp.bitwise_and(
                            lax.shift_right_logical(k, sh), 255), 4) + iota
                    if pp == 0:
                        plsc.addupdate_scatter(subhist, [slot], ones_i)
                    else:
                        act = lax.shift_right_logical(k, sh + 8) == prefix
                        plsc.addupdate_scatter(subhist, [slot], ones_i,
                                               mask=act)
                    return 0
                lax.fori_loop(0, NVEC, sc, 0, unroll=8)
                b, above = hist_walk(above, k120)
                prefix = jnp.left_shift(prefix, 8) | b
            T = prefix
            mG = above

            def extf(i, offs):
                goff, eoff = offs
                v = pvm[pl.ds(i * 16, 16)]
                k = lax.bitcast_convert_type(v, jnp.int32)
                gm = k > T
                em = jnp.logical_and(k == T, eoff < CAND - 32)
                gmi = gm.astype(jnp.int32)
                emi = em.astype(jnp.int32)
                gexc = plsc.cumsum(gmi) - gmi
                eexc = plsc.cumsum(emi) - emi
                idxv = i * 16 + iota
                plsc.store_scatter(cand, [goff + gexc], idxv, mask=gm)
                plsc.store_scatter(cand, [eoff + eexc], idxv, mask=em)
                return (goff + plsc.all_reduce_population_count(gm),
                        eoff + plsc.all_reduce_population_count(em))
            lax.fori_loop(0, NVEC, extf, (zero_i, mG), unroll=4)
            return jnp.int32(0)

        ok = jnp.logical_and(acnt_s >= K_SEL, acnt_s <= CAPA - 32)
        lax.cond(ok, fastp, slowp, 0)

        candv = [cand[pl.ds(16 * v_, 16)] for v_ in range(NCV)]
        kv0 = tuple(plsc.load_gather(pvm, [cv]) for cv in candv)
        outidx[pl.ds(112, 16)] = zero_i

        def select(j, kv):
            mx = kv[0]
            for v_ in range(1, NCV):
                mx = jnp.maximum(mx, kv[v_])
            m = jnp.max(mx)
            sel_v = zero_i
            sel_f = zero_i
            for v_ in range(NCV - 1, -1, -1):
                eq = kv[v_] == m
                hit = plsc.all_reduce_population_count(eq) > 0
                fv = plsc.all_reduce_ffs(eq)
                sel_v = jnp.where(hit, jnp.int32(v_), sel_v)
                sel_f = jnp.where(hit, fv, sel_f)
            idx_row = zero_i
            for v_ in range(NCV):
                idx_row = jnp.where(sel_v == v_, candv[v_], idx_row)
            idx_sel = jnp.sum(jnp.where(iota == sel_f, idx_row, 0))
            lane0 = iota == 0
            jb = jnp.broadcast_to(j, (16,))
            plsc.store_scatter(outsc, [jb], jnp.broadcast_to(m, (16,)),
                               mask=lane0)
            plsc.store_scatter(outidx, [jb], jnp.broadcast_to(idx_sel, (16,)),
                               mask=lane0)
            lanehit = iota == sel_f
            return tuple(
                jnp.where(jnp.logical_and(sel_v == v_, lanehit), -1.0, kv[v_])
                for v_ in range(NCV))
        lax.fori_loop(0, K_SEL, select, kv0)

        sv = scvm[pl.ds(0, 16)]
        sw = jnp.sum(jnp.where(iota == 0, sv, 0.0))
        sh_ = jnp.sum(jnp.where(iota == 1, sv, 0.0))
        for v_ in range(8):
            idxv = outidx[pl.ds(16 * v_, 16)]
            qv = lax.div(idxv, jnp.int32(C))
            outlb[pl.ds(16 * v_, 16)] = idxv - qv * C
            cx = plsc.load_gather(bxvm, [qv, zero_i])
            cy = plsc.load_gather(bxvm, [qv, zero_i + 1])
            w = plsc.load_gather(bxvm, [qv, zero_i + 2])
            h = plsc.load_gather(bxvm, [qv, zero_i + 3])
            rows = 16 * v_ + iota
            mrow = rows < K_SEL
            plsc.store_scatter(outbx, [rows, zero_i], (cx - 0.5 * w) * sw,
                               mask=mrow)
            plsc.store_scatter(outbx, [rows, zero_i + 1], (cy - 0.5 * h) * sh_,
                               mask=mrow)
            plsc.store_scatter(outbx, [rows, zero_i + 2], (cx + 0.5 * w) * sw,
                               mask=mrow)
            plsc.store_scatter(outbx, [rows, zero_i + 3], (cy + 0.5 * h) * sh_,
                               mask=mrow)

        pltpu.sync_copy(outsc.at[pl.ds(0, K_SEL)], scores_hbm.at[img])
        pltpu.sync_copy(outlb.at[pl.ds(0, K_SEL)], labels_hbm.at[img])
        pltpu.sync_copy(outbx, oboxes_hbm.at[img])
        return 0

    lax.fori_loop(0, IMG_PER_W, per_image, 0)


@jax.jit
def kernel(pred_logits, pred_boxes, target_sizes):
    prob = jax.nn.sigmoid(pred_logits).reshape(B, N)
    ts = target_sizes.astype(jnp.float32)
    scale = jnp.stack([ts[:, 1], ts[:, 0], ts[:, 1], ts[:, 0]], axis=1)

    mesh = plsc.VectorSubcoreMesh(
        core_axis_name="c", subcore_axis_name="s",
        num_cores=NC, num_subcores=NS)
    run = pl.kernel(
        _topk_body,
        out_type=(
            jax.ShapeDtypeStruct((B, K_SEL), jnp.float32),
            jax.ShapeDtypeStruct((B, K_SEL), jnp.int32),
            jax.ShapeDtypeStruct((B, K_SEL, 4), jnp.float32),
        ),
        mesh=mesh,
        compiler_params=pltpu.CompilerParams(
            needs_layout_passes=False, use_tc_tiling_on_sc=False),
        scratch_types=[
            pltpu.VMEM((NPAD,), jnp.float32),
            pltpu.VMEM((Q, 4), jnp.float32),
            pltpu.VMEM((16,), jnp.float32),
            pltpu.VMEM((4096,), jnp.int32),
            pltpu.VMEM((CAND,), jnp.int32),
            pltpu.VMEM((CAPA,), jnp.int32),
            pltpu.VMEM((CAPA,), jnp.int32),
            pltpu.VMEM((128,), jnp.float32),
            pltpu.VMEM((128,), jnp.int32),
            pltpu.VMEM((128,), jnp.int32),
            pltpu.VMEM((K_SEL, 4), jnp.float32),
        ],
    )
    scores, labels, boxes = run(prob, pred_boxes, scale)
    return scores, labels, boxes


# speedup vs baseline: 1.0891x; 1.0891x over previous
import jax
import jax.numpy as jnp
from jax import lax
from jax.experimental import pallas as pl
from jax.experimental.pallas import tpu as pltpu
from jax.experimental.pallas import tpu_sc as plsc

B = 128
Q = 900
C = 91
N = Q * C
NPAD = 81920
NVEC = NPAD // 16
NG = NPAD // 128
TAIL0 = (N // 16) * 16
NREAL = N - TAIL0
K_SEL = 120
NC = 2
NS = 16
NW = NC * NS
IMG_PER_W = B // NW
CAND = 160
NCV = CAND // 16
CAPA = 8192
NSAMP = 80
K_G = 12


def _topk_body(prob_hbm, boxes_hbm, scale_hbm,
               scores_hbm, labels_hbm, oboxes_hbm,
               pvm, bxvm, scvm, subhist, cand, aibuf, avbuf,
               outsc, outidx, outlb, outbx):
    wid = lax.axis_index("s") * NC + lax.axis_index("c")
    iota = lax.iota(jnp.int32, 16)
    ones_i = jnp.ones((16,), jnp.int32)
    zero_i = jnp.zeros((16,), jnp.int32)
    k120 = jnp.full((16,), K_SEL, jnp.int32)
    kg = jnp.full((16,), K_G, jnp.int32)

    def zero_hist():
        def zh(z, _):
            subhist[pl.ds(z * 16, 16)] = zero_i
            return 0
        lax.fori_loop(0, 256, zh, 0, unroll=8)

    def hist_walk(above_in, target):
        need = target - above_in

        def walk(c2, carry):
            found, bsel, above, csum = carry
            c = 15 - c2
            base = 256 * c
            h = zero_i
            for l in range(16):
                h = h + plsc.load_gather(
                    subhist, [base + jnp.left_shift(iota, 4) + l])
            rh = lax.rev(h, (0,))
            cs = plsc.cumsum(rh)
            tot = jnp.sum(h)
            contains = jnp.logical_and(jnp.logical_not(found),
                                       csum + tot >= need)
            mvec = (csum + cs) >= need
            r = plsc.all_reduce_ffs(mvec)
            b_here = 16 * c + 15 - r
            above_here = above + csum + jnp.sum(jnp.where(iota < r, rh, 0))
            found2 = jnp.logical_or(found, contains)
            bsel2 = jnp.where(contains, b_here, bsel)
            above2 = jnp.where(contains, above_here, above)
            return found2, bsel2, above2, csum + tot
        found0 = jnp.zeros((16,), jnp.bool_)
        _, bsel, above_out, _ = lax.fori_loop(
            0, 16, walk, (found0, zero_i, above_in, zero_i))
        return bsel, above_out

    def per_image(t, _):
        img = wid * IMG_PER_W + t

        pltpu.sync_copy(prob_hbm.at[img], pvm.at[pl.ds(0, N)])
        pltpu.sync_copy(boxes_hbm.at[img], bxvm)
        pltpu.sync_copy(scale_hbm.at[img], scvm.at[pl.ds(0, 4)])
        tl = pvm[pl.ds(TAIL0, 16)]
        pvm[pl.ds(TAIL0, 16)] = jnp.where(iota < NREAL, tl, 0.0)
        pvm[pl.ds(TAIL0 + 16, 16)] = jnp.zeros((16,), jnp.float32)
        for v_ in range(NCV):
            cand[pl.ds(16 * v_, 16)] = jnp.full((16,), N, jnp.int32)

        zero_hist()

        def samp1(i, _):
            v = pvm[pl.ds(i * 1024, 16)]
            k = lax.bitcast_convert_type(v, jnp.int32)
            avbuf[pl.ds(i * 16, 16)] = k
            slot = jnp.left_shift(lax.shift_right_logical(k, 24), 4) + iota
            plsc.addupdate_scatter(subhist, [slot], ones_i)
            return 0
        lax.fori_loop(0, NSAMP, samp1, 0, unroll=8)
        b1s, a1s = hist_walk(zero_i, kg)
        zero_hist()

        def samp2(i, _):
            k = avbuf[pl.ds(i * 16, 16)]
            act = lax.shift_right_logical(k, 24) == b1s
            slot = jnp.left_shift(
                jnp.bitwise_and(lax.shift_right_logical(k, 16), 255), 4) + iota
            plsc.addupdate_scatter(subhist, [slot], ones_i, mask=act)
            return 0
        lax.fori_loop(0, NSAMP, samp2, 0, unroll=8)
        b2s, _ = hist_walk(a1s, kg)
        tg = jnp.left_shift(jnp.left_shift(b1s, 8) | b2s, 16)

        def scanm(i, aoff):
            v = pvm[pl.ds(i, 16)]
            k = lax.bitcast_convert_type(v, jnp.int32)
            am0 = k >= tg
            am = jnp.logical_and(am0, aoff < CAPA - 16)
            ami = am.astype(jnp.int32)
            aexc = plsc.cumsum(ami) - ami
            plsc.store_scatter(aibuf, [aoff + aexc], i + iota, mask=am)
            plsc.store_scatter(avbuf, [aoff + aexc], k, mask=am)
            return aoff + plsc.all_reduce_population_count(am0)
        acnt = plsc.parallel_loop(
            0, NPAD, step=16, unroll=8, carry=zero_i)(scanm)
        acnt_s = jnp.max(acnt)

        def fastp(_):
            nv = lax.div(acnt_s + 15, jnp.int32(16))

            def buf_pass(sh, prefix, above):
                zero_hist()

                def sc(i, _, sh=sh, prefix=prefix):
                    av = avbuf[pl.ds(i * 16, 16)]
                    valid = (i * 16 + iota) < acnt
                    if prefix is None:
                        act = valid
                    else:
                        act = jnp.logical_and(
                            valid,
                            lax.shift_right_logical(av, sh + 8) == prefix)
                    slot = jnp.left_shift(
                        jnp.bitwise_and(
                            lax.shift_right_logical(av, sh), 255), 4) + iota
                    plsc.addupdate_scatter(subhist, [slot], ones_i, mask=act)
                    return 0
                lax.fori_loop(0, nv, sc, 0)
                return hist_walk(above, k120)

            b1, a1 = buf_pass(24, None, zero_i)
            b2, a2 = buf_pass(16, b1, a1)
            pre16 = jnp.left_shift(b1, 8) | b2
            b3, a3 = buf_pass(8, pre16, a2)
            pre24 = jnp.left_shift(pre16, 8) | b3
            b4, mG = buf_pass(0, pre24, a3)
            T = jnp.left_shift(pre24, 8) | b4

            def ext(i, offs):
                goff, eoff = offs
                av = avbuf[pl.ds(i * 16, 16)]
                ai = aibuf[pl.ds(i * 16, 16)]
                valid = (i * 16 + iota) < acnt
                gm = jnp.logical_and(valid, av > T)
                em = jnp.logical_and(jnp.logical_and(valid, av == T),
                                     eoff < CAND - 32)
                gmi = gm.astype(jnp.int32)
                emi = em.astype(jnp.int32)
                gexc = plsc.cumsum(gmi) - gmi
                eexc = plsc.cumsum(emi) - emi
                plsc.store_scatter(cand, [goff + gexc], ai, mask=gm)
                plsc.store_scatter(cand, [eoff + eexc], ai, mask=em)
                return (goff + plsc.all_reduce_population_count(gm),
                        eoff + plsc.all_reduce_population_count(em))
            lax.fori_loop(0, nv, ext, (zero_i, mG))
            return jnp.int32(0)

        def slowp(_):
            prefix = zero_i
            above = zero_i
            for pp in range(4):
                sh = 24 - 8 * pp
                zero_hist()

                def sc(i, _, sh=sh, pp=pp, prefix=prefix):
                    v = pvm[pl.ds(i * 16, 16)]
                    k = lax.bitcast_convert_type(v, jnp.int32)
                    slot = jnp.left_shift(
                        jnp.bitwise_and(
                            lax.shift_right_logical(k, sh), 255), 4) + iota
                    if pp == 0:
                        plsc.addupdate_scatter(subhist, [slot], ones_i)
                    else:
                        act = lax.shift_right_logical(k, sh + 8) == prefix
                        plsc.addupdate_scatter(subhist, [slot], ones_i,
                                               mask=act)
                    return 0
                lax.fori_loop(0, NVEC, sc, 0, unroll=8)
                b, above = hist_walk(above, k120)
                prefix = jnp.left_shift(prefix, 8) | b
            T = prefix
            mG = above

            def extf(i, offs):
                goff, eoff = offs
                v = pvm[pl.ds(i * 16, 16)]
                k = lax.bitcast_convert_type(v, jnp.int32)
                gm = k > T
                em = jnp.logical_and(k == T, eoff < CAND - 32)
                gmi = gm.astype(jnp.int32)
                emi = em.astype(jnp.int32)
                gexc = plsc.cumsum(gmi) - gmi
                eexc = plsc.cumsum(emi) - emi
                idxv = i * 16 + iota
                plsc.store_scatter(cand, [goff + gexc], idxv, mask=gm)
                plsc.store_scatter(cand, [eoff + eexc], idxv, mask=em)
                return (goff + plsc.all_reduce_population_count(gm),
                        eoff + plsc.all_reduce_population_count(em))
            lax.fori_loop(0, NVEC, extf, (zero_i, mG), unroll=4)
            return jnp.int32(0)

        ok = jnp.logical_and(acnt_s >= K_SEL, acnt_s <= CAPA - 32)
        lax.cond(ok, fastp, slowp, 0)

        candv = [cand[pl.ds(16 * v_, 16)] for v_ in range(NCV)]
        kv0 = tuple(plsc.load_gather(pvm, [cv]) for cv in candv)
        outidx[pl.ds(112, 16)] = zero_i

        def select(j, kv):
            mx = kv[0]
            for v_ in range(1, NCV):
                mx = jnp.maximum(mx, kv[v_])
            m = jnp.max(mx)
            sel_v = zero_i
            sel_f = zero_i
            for v_ in range(NCV - 1, -1, -1):
                eq = kv[v_] == m
                hit = plsc.all_reduce_population_count(eq) > 0
                fv = plsc.all_reduce_ffs(eq)
                sel_v = jnp.where(hit, jnp.int32(v_), sel_v)
                sel_f = jnp.where(hit, fv, sel_f)
            idx_row = zero_i
            for v_ in range(NCV):
                idx_row = jnp.where(sel_v == v_, candv[v_], idx_row)
            idx_sel = jnp.sum(jnp.where(iota == sel_f, idx_row, 0))
            lane0 = iota == 0
            jb = jnp.broadcast_to(j, (16,))
            plsc.store_scatter(outsc, [jb], jnp.broadcast_to(m, (16,)),
                               mask=lane0)
            plsc.store_scatter(outidx, [jb], jnp.broadcast_to(idx_sel, (16,)),
                               mask=lane0)
            lanehit = iota == sel_f
            return tuple(
                jnp.where(jnp.logical_and(sel_v == v_, lanehit), -1.0, kv[v_])
                for v_ in range(NCV))
        lax.fori_loop(0, K_SEL, select, kv0)

        sv = scvm[pl.ds(0, 16)]
        sw = jnp.sum(jnp.where(iota == 0, sv, 0.0))
        sh_ = jnp.sum(jnp.where(iota == 1, sv, 0.0))
        for v_ in range(8):
            idxv = outidx[pl.ds(16 * v_, 16)]
            qv = lax.div(idxv, jnp.int32(C))
            outlb[pl.ds(16 * v_, 16)] = idxv - qv * C
            cx = plsc.load_gather(bxvm, [qv, zero_i])
            cy = plsc.load_gather(bxvm, [qv, zero_i + 1])
            w = plsc.load_gather(bxvm, [qv, zero_i + 2])
            h = plsc.load_gather(bxvm, [qv, zero_i + 3])
            rows = 16 * v_ + iota
            mrow = rows < K_SEL
            plsc.store_scatter(outbx, [rows, zero_i], (cx - 0.5 * w) * sw,
                               mask=mrow)
            plsc.store_scatter(outbx, [rows, zero_i + 1], (cy - 0.5 * h) * sh_,
                               mask=mrow)
            plsc.store_scatter(outbx, [rows, zero_i + 2], (cx + 0.5 * w) * sw,
                               mask=mrow)
            plsc.store_scatter(outbx, [rows, zero_i + 3], (cy + 0.5 * h) * sh_,
                               mask=mrow)

        pltpu.sync_copy(outsc.at[pl.ds(0, K_SEL)], scores_hbm.at[img])
        pltpu.sync_copy(outlb.at[pl.ds(0, K_SEL)], labels_hbm.at[img])
        pltpu.sync_copy(outbx, oboxes_hbm.at[img])
        return 0

    lax.fori_loop(0, IMG_PER_W, per_image, 0)


@jax.jit
def kernel(pred_logits, pred_boxes, target_sizes):
    prob = jax.nn.sigmoid(pred_logits).reshape(B, N)
    ts = target_sizes.astype(jnp.float32)
    scale = jnp.stack([ts[:, 1], ts[:, 0], ts[:, 1], ts[:, 0]], axis=1)

    mesh = plsc.VectorSubcoreMesh(
        core_axis_name="c", subcore_axis_name="s",
        num_cores=NC, num_subcores=NS)
    run = pl.kernel(
        _topk_body,
        out_type=(
            jax.ShapeDtypeStruct((B, K_SEL), jnp.float32),
            jax.ShapeDtypeStruct((B, K_SEL), jnp.int32),
            jax.ShapeDtypeStruct((B, K_SEL, 4), jnp.float32),
        ),
        mesh=mesh,
        compiler_params=pltpu.CompilerParams(
            needs_layout_passes=False, use_tc_tiling_on_sc=False),
        scratch_types=[
            pltpu.VMEM((NPAD,), jnp.float32),
            pltpu.VMEM((Q, 4), jnp.float32),
            pltpu.VMEM((16,), jnp.float32),
            pltpu.VMEM((4096,), jnp.int32),
            pltpu.VMEM((CAND,), jnp.int32),
            pltpu.VMEM((CAPA,), jnp.int32),
            pltpu.VMEM((CAPA,), jnp.int32),
            pltpu.VMEM((128,), jnp.float32),
            pltpu.VMEM((128,), jnp.int32),
            pltpu.VMEM((128,), jnp.int32),
            pltpu.VMEM((K_SEL, 4), jnp.float32),
        ],
    )
    scores, labels, boxes = run(prob, pred_boxes, scale)
    return scores, labels, boxes
